# baseline (speedup 1.0000x reference)
import functools

import jax
import jax.numpy as jnp
from jax import lax
from jax.experimental import pallas as pl
from jax.experimental.pallas import tpu as pltpu
from jax.experimental.pallas import tpu_sc as plsc

_CUT1 = 20000
_CUT2 = 60000
_D0, _D1, _D2 = 1024, 256, 64
_ED = 1024
_NC, _NS = 2, 16
_NW = _NC * _NS
_TOK = 4 * 2048
_TPW = _TOK // _NW
_C0 = 64
_C12 = 128


def _sc_gather_body(x_hbm, e0_hbm, e1_hbm, e2_hbm,
                    g0_hbm, g1_hbm, g2_hbm,
                    xv, i0, i1, i2, buf0, buf1, buf2, sem):
    c = lax.axis_index("c")
    s = lax.axis_index("s")
    wid = s * _NC + c
    base = wid * _TPW
    pltpu.sync_copy(x_hbm.at[pl.ds(base, _TPW)], xv)
    for i in range(_TPW // 16):
        v = xv[pl.ds(i * 16, 16)]
        zero = jnp.zeros((16,), jnp.int32)
        r0 = jnp.where(v < _CUT1, v, zero)
        in1 = jnp.logical_and(v >= _CUT1, v < _CUT2)
        r1 = jnp.where(in1, v - _CUT1, zero)
        r2 = jnp.where(v >= _CUT2, lax.shift_right_logical(v - _CUT2, 1), zero)
        ci0 = i // (_C0 // 16)
        of0 = (i % (_C0 // 16)) * 16
        i0[ci0, pl.ds(of0, 16)] = r0
        ci1 = i // (_C12 // 16)
        of1 = (i % (_C12 // 16)) * 16
        i1[ci1, pl.ds(of1, 16)] = r1
        i2[ci1, pl.ds(of1, 16)] = r2
    for ci in range(_TPW // _C0):
        pltpu.async_copy(e0_hbm.at[i0.at[ci]], buf0, sem).wait()
        pltpu.sync_copy(buf0, g0_hbm.at[pl.ds(base + ci * _C0, _C0)])
    for ci in range(_TPW // _C12):
        pltpu.async_copy(e1_hbm.at[i1.at[ci]], buf1, sem).wait()
        pltpu.sync_copy(buf1, g1_hbm.at[pl.ds(base + ci * _C12, _C12)])
    for ci in range(_TPW // _C12):
        pltpu.async_copy(e2_hbm.at[i2.at[ci]], buf2, sem).wait()
        pltpu.sync_copy(buf2, g2_hbm.at[pl.ds(base + ci * _C12, _C12)])


@jax.jit
def _sc_gather(xf, e0, e1, e2):
    mesh = plsc.VectorSubcoreMesh(core_axis_name="c", subcore_axis_name="s")
    return pl.kernel(
        _sc_gather_body,
        out_type=(
            jax.ShapeDtypeStruct((_TOK, _D0), jnp.float32),
            jax.ShapeDtypeStruct((_TOK, _D1), jnp.float32),
            jax.ShapeDtypeStruct((_TOK, 2 * _D2), jnp.float32),
        ),
        mesh=mesh,
        scratch_types=[
            pltpu.VMEM((_TPW,), jnp.int32),
            pltpu.VMEM((_TPW // _C0, _C0), jnp.int32),
            pltpu.VMEM((_TPW // _C12, _C12), jnp.int32),
            pltpu.VMEM((_TPW // _C12, _C12), jnp.int32),
            pltpu.VMEM((_C0, _D0), jnp.float32),
            pltpu.VMEM((_C12, _D1), jnp.float32),
            pltpu.VMEM((_C12, 2 * _D2), jnp.float32),
            pltpu.SemaphoreType.DMA,
        ],
    )(xf, e0, e1, e2)


_BT = 512


def _tc_body(x_ref, g0_ref, g1_ref, g2_ref, w0_ref, w1_ref, w2_ref, o_ref):
    xv = x_ref[...]
    m1 = jnp.logical_and(xv >= _CUT1, xv < _CUT2).astype(jnp.float32)
    m2 = (xv >= _CUT2).astype(jnp.float32)
    odd = jnp.bitwise_and(xv - _CUT2, 1).astype(jnp.float32)
    col = lax.broadcasted_iota(jnp.int32, (_BT, 2 * _D2), 1)
    half_m = jnp.where(col < _D2, m2 * (1.0 - odd), m2 * odd)
    acc = jnp.dot(g0_ref[...], w0_ref[...], preferred_element_type=jnp.float32)
    acc = acc + jnp.dot(g1_ref[...] * m1, w1_ref[...],
                        preferred_element_type=jnp.float32)
    acc = acc + jnp.dot(g2_ref[...] * half_m, w2_ref[...],
                        preferred_element_type=jnp.float32)
    o_ref[...] = acc


@jax.jit
def _tc_matmul(x2d, g0, g1, g2, w0, w1, w2):
    grid = (_TOK // _BT,)
    return pl.pallas_call(
        _tc_body,
        grid=grid,
        in_specs=[
            pl.BlockSpec((_BT, 1), lambda i: (i, 0)),
            pl.BlockSpec((_BT, _D0), lambda i: (i, 0)),
            pl.BlockSpec((_BT, _D1), lambda i: (i, 0)),
            pl.BlockSpec((_BT, 2 * _D2), lambda i: (i, 0)),
            pl.BlockSpec((_D0, _ED), lambda i: (0, 0)),
            pl.BlockSpec((_D1, _ED), lambda i: (0, 0)),
            pl.BlockSpec((2 * _D2, _ED), lambda i: (0, 0)),
        ],
        out_specs=pl.BlockSpec((_BT, _ED), lambda i: (i, 0)),
        out_shape=jax.ShapeDtypeStruct((_TOK, _ED), jnp.float32),
        compiler_params=pltpu.CompilerParams(
            dimension_semantics=("arbitrary",),
        ),
    )(x2d, g0, g1, g2, w0, w1, w2)


def kernel(x, E0, W0, E1, W1, E2, W2):
    xf = x.reshape(-1)
    e2p = E2.reshape(-1, 2 * _D2)
    w2s = jnp.concatenate([W2, W2], axis=0)
    g0, g1, g2 = _sc_gather(xf, E0, E1, e2p)
    out = _tc_matmul(xf.reshape(-1, 1), g0, g1, g2, W0, W1, w2s)
    return out.reshape(x.shape + (_ED,))


# speedup vs baseline: 1.4612x; 1.4612x over previous
import functools

import jax
import jax.numpy as jnp
from jax import lax
from jax.experimental import pallas as pl
from jax.experimental.pallas import tpu as pltpu
from jax.experimental.pallas import tpu_sc as plsc

_CUT1 = 20000
_CUT2 = 60000
_D0, _D1, _D2 = 1024, 256, 64
_ED = 1024
_NC, _NS = 2, 16
_NW = _NC * _NS
_TOK = 4 * 2048
_TPW = _TOK // _NW
_C0 = 64
_C12 = 128


_NCH = (8, 4, 4)
_CH = (32, 64, 64)
_DW = (_D0, _D1, 2 * _D2)


def _sc_gather_body(x_hbm, e0_hbm, e1_hbm, e2_hbm,
                    g0_hbm, g1_hbm, g2_hbm, xv, *scr):
    c = lax.axis_index("c")
    s = lax.axis_index("s")
    wid = s * _NC + c
    base = wid * _TPW
    n_idx = sum(_NCH)
    idx = [list(scr[:_NCH[0]]),
           list(scr[_NCH[0]:_NCH[0] + _NCH[1]]),
           list(scr[_NCH[0] + _NCH[1]:n_idx])]
    bufs = [list(scr[n_idx:n_idx + 2]),
            list(scr[n_idx + 2:n_idx + 4]),
            list(scr[n_idx + 4:n_idx + 6])]
    sems = [list(scr[n_idx + 6:n_idx + 8]),
            list(scr[n_idx + 8:n_idx + 10]),
            list(scr[n_idx + 10:n_idx + 12])]
    tables = (e0_hbm, e1_hbm, e2_hbm)
    gouts = (g0_hbm, g1_hbm, g2_hbm)

    pltpu.sync_copy(x_hbm.at[pl.ds(base, _TPW)], xv)
    for i in range(_TPW // 16):
        v = xv[pl.ds(i * 16, 16)]
        zero = jnp.zeros((16,), jnp.int32)
        r0 = jnp.where(v < _CUT1, v, zero)
        in1 = jnp.logical_and(v >= _CUT1, v < _CUT2)
        r1 = jnp.where(in1, v - _CUT1, zero)
        r2 = jnp.where(v >= _CUT2, lax.shift_right_logical(v - _CUT2, 1), zero)
        for t, r in ((0, r0), (1, r1), (2, r2)):
            per = _CH[t] // 16
            idx[t][i // per][pl.ds((i % per) * 16, 16)] = r

    def mk_gather(t, ci):
        sl = ci % 2
        return pltpu.make_async_copy(
            tables[t].at[idx[t][ci]], bufs[t][sl], sems[t][sl])

    def mk_scatter(t, ci):
        sl = ci % 2
        return pltpu.make_async_copy(
            bufs[t][sl], gouts[t].at[pl.ds(base + ci * _CH[t], _CH[t])],
            sems[t][sl])

    gathers = {}
    scatters = {}
    for t in range(3):
        for ci in range(min(2, _NCH[t])):
            gathers[(t, ci)] = mk_gather(t, ci)
            gathers[(t, ci)].start()
    rounds = max(n // 2 for n in _NCH)
    for r in range(rounds):
        for t in range(3):
            per_round = _NCH[t] // rounds
            for k in range(per_round):
                ci = r * per_round + k
                gathers[(t, ci)].wait()
                sc = mk_scatter(t, ci)
                scatters[(t, ci)] = sc
                sc.start()
                if ci + 2 < _NCH[t]:
                    sc.wait()
                    scatters.pop((t, ci))
                    gathers[(t, ci + 2)] = mk_gather(t, ci + 2)
                    gathers[(t, ci + 2)].start()
    for sc in scatters.values():
        sc.wait()


@jax.jit
def _sc_gather(xf, e0, e1, e2):
    mesh = plsc.VectorSubcoreMesh(core_axis_name="c", subcore_axis_name="s")
    scratch = [pltpu.VMEM((_TPW,), jnp.int32)]
    scratch = ([pltpu.VMEM((_CH[t],), jnp.int32)
                for t in range(3) for _ in range(_NCH[t])]
               + [pltpu.VMEM((_CH[t], _DW[t]), jnp.float32)
                  for t in range(3) for _ in range(2)]
               + [pltpu.SemaphoreType.DMA for _ in range(6)])
    return pl.kernel(
        _sc_gather_body,
        out_type=(
            jax.ShapeDtypeStruct((_TOK, _D0), jnp.float32),
            jax.ShapeDtypeStruct((_TOK, _D1), jnp.float32),
            jax.ShapeDtypeStruct((_TOK, 2 * _D2), jnp.float32),
        ),
        mesh=mesh,
        scratch_types=[pltpu.VMEM((_TPW,), jnp.int32)] + scratch,
    )(xf, e0, e1, e2)


_BT = 512


def _tc_body(x_ref, g0_ref, g1_ref, g2_ref, w0_ref, w1_ref, w2_ref, o_ref):
    xv = x_ref[...]
    m1 = jnp.logical_and(xv >= _CUT1, xv < _CUT2).astype(jnp.float32)
    m2 = (xv >= _CUT2).astype(jnp.float32)
    odd = jnp.bitwise_and(xv - _CUT2, 1).astype(jnp.float32)
    col = lax.broadcasted_iota(jnp.int32, (_BT, 2 * _D2), 1)
    half_m = jnp.where(col < _D2, m2 * (1.0 - odd), m2 * odd)
    acc = jnp.dot(g0_ref[...], w0_ref[...], preferred_element_type=jnp.float32)
    acc = acc + jnp.dot(g1_ref[...] * m1, w1_ref[...],
                        preferred_element_type=jnp.float32)
    acc = acc + jnp.dot(g2_ref[...] * half_m, w2_ref[...],
                        preferred_element_type=jnp.float32)
    o_ref[...] = acc


@jax.jit
def _tc_matmul(x2d, g0, g1, g2, w0, w1, w2):
    grid = (_TOK // _BT,)
    return pl.pallas_call(
        _tc_body,
        grid=grid,
        in_specs=[
            pl.BlockSpec((_BT, 1), lambda i: (i, 0)),
            pl.BlockSpec((_BT, _D0), lambda i: (i, 0)),
            pl.BlockSpec((_BT, _D1), lambda i: (i, 0)),
            pl.BlockSpec((_BT, 2 * _D2), lambda i: (i, 0)),
            pl.BlockSpec((_D0, _ED), lambda i: (0, 0)),
            pl.BlockSpec((_D1, _ED), lambda i: (0, 0)),
            pl.BlockSpec((2 * _D2, _ED), lambda i: (0, 0)),
        ],
        out_specs=pl.BlockSpec((_BT, _ED), lambda i: (i, 0)),
        out_shape=jax.ShapeDtypeStruct((_TOK, _ED), jnp.float32),
        compiler_params=pltpu.CompilerParams(
            dimension_semantics=("arbitrary",),
        ),
    )(x2d, g0, g1, g2, w0, w1, w2)


def kernel(x, E0, W0, E1, W1, E2, W2):
    xf = x.reshape(-1)
    e2p = E2.reshape(-1, 2 * _D2)
    w2s = jnp.concatenate([W2, W2], axis=0)
    g0, g1, g2 = _sc_gather(xf, E0, E1, e2p)
    out = _tc_matmul(xf.reshape(-1, 1), g0, g1, g2, W0, W1, w2s)
    return out.reshape(x.shape + (_ED,))


# speedup vs baseline: 5.6621x; 3.8749x over previous
import functools

import jax
import jax.numpy as jnp
from jax import lax
from jax.experimental import pallas as pl
from jax.experimental.pallas import tpu as pltpu
from jax.experimental.pallas import tpu_sc as plsc

_CUT1 = 20000
_CUT2 = 60000
_D0, _D1, _D2 = 1024, 256, 64
_ED = 1024
_NC, _NS = 2, 16
_NW = _NC * _NS
_TOK = 4 * 2048
_TPW = _TOK // _NW
_C0 = 64
_C12 = 128


_NCH = (8, 4, 4)
_CH = (32, 64, 64)
_DW = (_D0, _D1, 2 * _D2)


def _sc_gather_body(x_hbm, e0_hbm, e1_hbm, e2_hbm,
                    g0_hbm, g1_hbm, g2_hbm, xv, *scr):
    c = lax.axis_index("c")
    s = lax.axis_index("s")
    wid = s * _NC + c
    base = wid * _TPW
    n_idx = sum(_NCH)
    idx = [list(scr[:_NCH[0]]),
           list(scr[_NCH[0]:_NCH[0] + _NCH[1]]),
           list(scr[_NCH[0] + _NCH[1]:n_idx])]
    bufs = [list(scr[n_idx:n_idx + 2]),
            list(scr[n_idx + 2:n_idx + 4]),
            list(scr[n_idx + 4:n_idx + 6])]
    sems = [list(scr[n_idx + 6:n_idx + 8]),
            list(scr[n_idx + 8:n_idx + 10]),
            list(scr[n_idx + 10:n_idx + 12])]
    tables = (e0_hbm, e1_hbm, e2_hbm)
    gouts = (g0_hbm, g1_hbm, g2_hbm)

    pltpu.sync_copy(x_hbm.at[pl.ds(base, _TPW)], xv)
    for i in range(_TPW // 16):
        v = xv[pl.ds(i * 16, 16)]
        r0 = lax.rem(v, jnp.full((16,), _CUT1, jnp.int32))
        t12 = lax.rem(v + _CUT1, jnp.full((16,), _CUT2 - _CUT1, jnp.int32))
        r1 = t12
        r2 = lax.shift_right_logical(t12, 1)
        for t, r in ((0, r0), (1, r1), (2, r2)):
            per = _CH[t] // 16
            idx[t][i // per][pl.ds((i % per) * 16, 16)] = r

    def mk_gather(t, ci):
        sl = ci % 2
        return pltpu.make_async_copy(
            tables[t].at[idx[t][ci]], bufs[t][sl], sems[t][sl])

    def mk_scatter(t, ci):
        sl = ci % 2
        return pltpu.make_async_copy(
            bufs[t][sl], gouts[t].at[pl.ds(base + ci * _CH[t], _CH[t])],
            sems[t][sl])

    gathers = {}
    scatters = {}
    for t in range(3):
        for ci in range(min(2, _NCH[t])):
            gathers[(t, ci)] = mk_gather(t, ci)
            gathers[(t, ci)].start()
    rounds = max(n // 2 for n in _NCH)
    for r in range(rounds):
        for t in range(3):
            per_round = _NCH[t] // rounds
            for k in range(per_round):
                ci = r * per_round + k
                gathers[(t, ci)].wait()
                sc = mk_scatter(t, ci)
                scatters[(t, ci)] = sc
                sc.start()
                if ci + 2 < _NCH[t]:
                    sc.wait()
                    scatters.pop((t, ci))
                    gathers[(t, ci + 2)] = mk_gather(t, ci + 2)
                    gathers[(t, ci + 2)].start()
    for sc in scatters.values():
        sc.wait()


@jax.jit
def _sc_gather(xf, e0, e1, e2):
    mesh = plsc.VectorSubcoreMesh(core_axis_name="c", subcore_axis_name="s")
    scratch = [pltpu.VMEM((_TPW,), jnp.int32)]
    scratch = ([pltpu.VMEM((_CH[t],), jnp.int32)
                for t in range(3) for _ in range(_NCH[t])]
               + [pltpu.VMEM((_CH[t], _DW[t]), jnp.float32)
                  for t in range(3) for _ in range(2)]
               + [pltpu.SemaphoreType.DMA for _ in range(6)])
    return pl.kernel(
        _sc_gather_body,
        out_type=(
            jax.ShapeDtypeStruct((_TOK, _D0), jnp.float32),
            jax.ShapeDtypeStruct((_TOK, _D1), jnp.float32),
            jax.ShapeDtypeStruct((_TOK, 2 * _D2), jnp.float32),
        ),
        mesh=mesh,
        scratch_types=[pltpu.VMEM((_TPW,), jnp.int32)] + scratch,
    )(xf, e0, e1, e2)


_BT = 512


def _tc_body(x_ref, g0_ref, g1_ref, g2_ref, w0_ref, w1_ref, w2_ref, o_ref):
    xv = x_ref[...]
    m0 = (xv < _CUT1).astype(jnp.float32)
    m1 = jnp.logical_and(xv >= _CUT1, xv < _CUT2).astype(jnp.float32)
    m2 = (xv >= _CUT2).astype(jnp.float32)
    odd = jnp.bitwise_and(xv - _CUT2, 1).astype(jnp.float32)
    col = lax.broadcasted_iota(jnp.int32, (_BT, 2 * _D2), 1)
    half_m = jnp.where(col < _D2, m2 * (1.0 - odd), m2 * odd)
    acc = jnp.dot(g0_ref[...] * m0, w0_ref[...],
                  preferred_element_type=jnp.float32)
    acc = acc + jnp.dot(g1_ref[...] * m1, w1_ref[...],
                        preferred_element_type=jnp.float32)
    acc = acc + jnp.dot(g2_ref[...] * half_m, w2_ref[...],
                        preferred_element_type=jnp.float32)
    o_ref[...] = acc


@jax.jit
def _tc_matmul(x2d, g0, g1, g2, w0, w1, w2):
    grid = (_TOK // _BT,)
    return pl.pallas_call(
        _tc_body,
        grid=grid,
        in_specs=[
            pl.BlockSpec((_BT, 1), lambda i: (i, 0)),
            pl.BlockSpec((_BT, _D0), lambda i: (i, 0)),
            pl.BlockSpec((_BT, _D1), lambda i: (i, 0)),
            pl.BlockSpec((_BT, 2 * _D2), lambda i: (i, 0)),
            pl.BlockSpec((_D0, _ED), lambda i: (0, 0)),
            pl.BlockSpec((_D1, _ED), lambda i: (0, 0)),
            pl.BlockSpec((2 * _D2, _ED), lambda i: (0, 0)),
        ],
        out_specs=pl.BlockSpec((_BT, _ED), lambda i: (i, 0)),
        out_shape=jax.ShapeDtypeStruct((_TOK, _ED), jnp.float32),
        compiler_params=pltpu.CompilerParams(
            dimension_semantics=("arbitrary",),
        ),
    )(x2d, g0, g1, g2, w0, w1, w2)


def kernel(x, E0, W0, E1, W1, E2, W2):
    xf = x.reshape(-1)
    e2p = E2.reshape(-1, 2 * _D2)
    w2s = jnp.concatenate([W2, W2], axis=0)
    g0, g1, g2 = _sc_gather(xf, E0, E1, e2p)
    out = _tc_matmul(xf.reshape(-1, 1), g0, g1, g2, W0, W1, w2s)
    return out.reshape(x.shape + (_ED,))


# speedup vs baseline: 5.6950x; 1.0058x over previous
import functools

import jax
import jax.numpy as jnp
from jax import lax
from jax.experimental import pallas as pl
from jax.experimental.pallas import tpu as pltpu
from jax.experimental.pallas import tpu_sc as plsc

_CUT1 = 20000
_CUT2 = 60000
_D0, _D1, _D2 = 1024, 256, 64
_ED = 1024
_NC, _NS = 2, 16
_NW = _NC * _NS
_TOK = 4 * 2048
_TPW = _TOK // _NW
_C0 = 64
_C12 = 128


_NCH = (8, 4, 4)
_CH = (32, 64, 64)
_DW = (_D0, _D1, 2 * _D2)


def _sc_gather_body(x_hbm, e0_hbm, e1_hbm, e2_hbm,
                    g0_hbm, g1_hbm, g2_hbm, xv, *scr):
    c = lax.axis_index("c")
    s = lax.axis_index("s")
    wid = s * _NC + c
    base = wid * _TPW
    n_idx = sum(_NCH)
    idx = [list(scr[:_NCH[0]]),
           list(scr[_NCH[0]:_NCH[0] + _NCH[1]]),
           list(scr[_NCH[0] + _NCH[1]:n_idx])]
    bufs = [list(scr[n_idx:n_idx + 2]),
            list(scr[n_idx + 2:n_idx + 4]),
            list(scr[n_idx + 4:n_idx + 6])]
    sems = [list(scr[n_idx + 6:n_idx + 8]),
            list(scr[n_idx + 8:n_idx + 10]),
            list(scr[n_idx + 10:n_idx + 12])]
    tables = (e0_hbm, e1_hbm, e2_hbm)
    gouts = (g0_hbm, g1_hbm, g2_hbm)

    pltpu.sync_copy(x_hbm.at[pl.ds(base, _TPW)], xv)
    for i in range(_TPW // 16):
        v = xv[pl.ds(i * 16, 16)]
        r0 = lax.rem(v, jnp.full((16,), _CUT1, jnp.int32))
        t12 = lax.rem(v + _CUT1, jnp.full((16,), _CUT2 - _CUT1, jnp.int32))
        r1 = t12
        r2 = lax.shift_right_logical(t12, 1)
        for t, r in ((0, r0), (1, r1), (2, r2)):
            per = _CH[t] // 16
            idx[t][i // per][pl.ds((i % per) * 16, 16)] = r

    def mk_gather(t, ci):
        sl = ci % 2
        return pltpu.make_async_copy(
            tables[t].at[idx[t][ci]], bufs[t][sl], sems[t][sl])

    def mk_scatter(t, ci):
        sl = ci % 2
        return pltpu.make_async_copy(
            bufs[t][sl], gouts[t].at[pl.ds(base + ci * _CH[t], _CH[t])],
            sems[t][sl])

    gathers = {}
    scatters = {}
    for t in range(3):
        for ci in range(min(2, _NCH[t])):
            gathers[(t, ci)] = mk_gather(t, ci)
            gathers[(t, ci)].start()
    rounds = max(n // 2 for n in _NCH)
    for r in range(rounds):
        for t in range(3):
            per_round = _NCH[t] // rounds
            for k in range(per_round):
                ci = r * per_round + k
                gathers[(t, ci)].wait()
                sc = mk_scatter(t, ci)
                scatters[(t, ci)] = sc
                sc.start()
                if ci + 2 < _NCH[t]:
                    sc.wait()
                    scatters.pop((t, ci))
                    gathers[(t, ci + 2)] = mk_gather(t, ci + 2)
                    gathers[(t, ci + 2)].start()
    for sc in scatters.values():
        sc.wait()


@jax.jit
def _sc_gather(xf, e0, e1, e2):
    mesh = plsc.VectorSubcoreMesh(core_axis_name="c", subcore_axis_name="s")
    scratch = [pltpu.VMEM((_TPW,), jnp.int32)]
    scratch = ([pltpu.VMEM((_CH[t],), jnp.int32)
                for t in range(3) for _ in range(_NCH[t])]
               + [pltpu.VMEM((_CH[t], _DW[t]), jnp.float32)
                  for t in range(3) for _ in range(2)]
               + [pltpu.SemaphoreType.DMA for _ in range(6)])
    return pl.kernel(
        _sc_gather_body,
        out_type=(
            jax.ShapeDtypeStruct((_TOK, _D0), jnp.float32),
            jax.ShapeDtypeStruct((_TOK, _D1), jnp.float32),
            jax.ShapeDtypeStruct((_TOK, 2 * _D2), jnp.float32),
        ),
        mesh=mesh,
        scratch_types=[pltpu.VMEM((_TPW,), jnp.int32)] + scratch,
    )(xf, e0, e1, e2)


_BT = 512


def _tc_body(x_ref, g0_ref, g1_ref, g2_ref, w0_ref, w1_ref, w2_ref, o_ref):
    xv = x_ref[...]
    bf = jnp.bfloat16
    m0 = (xv < _CUT1).astype(bf)
    m1 = jnp.logical_and(xv >= _CUT1, xv < _CUT2).astype(bf)
    m2 = (xv >= _CUT2).astype(bf)
    odd = jnp.bitwise_and(xv - _CUT2, 1)
    g2 = g2_ref[...].astype(bf)
    g2sel = jnp.where(odd > 0, g2[:, _D2:], g2[:, :_D2])
    acc = jnp.dot(g0_ref[...].astype(bf) * m0, w0_ref[...],
                  preferred_element_type=jnp.float32)
    acc = acc + jnp.dot(g1_ref[...].astype(bf) * m1, w1_ref[...],
                        preferred_element_type=jnp.float32)
    acc = acc + jnp.dot(g2sel * m2, w2_ref[...],
                        preferred_element_type=jnp.float32)
    o_ref[...] = acc


@jax.jit
def _tc_matmul(x2d, g0, g1, g2, w0, w1, w2):
    grid = (_TOK // _BT,)
    return pl.pallas_call(
        _tc_body,
        grid=grid,
        in_specs=[
            pl.BlockSpec((_BT, 1), lambda i: (i, 0)),
            pl.BlockSpec((_BT, _D0), lambda i: (i, 0)),
            pl.BlockSpec((_BT, _D1), lambda i: (i, 0)),
            pl.BlockSpec((_BT, 2 * _D2), lambda i: (i, 0)),
            pl.BlockSpec((_D0, _ED), lambda i: (0, 0)),
            pl.BlockSpec((_D1, _ED), lambda i: (0, 0)),
            pl.BlockSpec((_D2, _ED), lambda i: (0, 0)),
        ],
        out_specs=pl.BlockSpec((_BT, _ED), lambda i: (i, 0)),
        out_shape=jax.ShapeDtypeStruct((_TOK, _ED), jnp.float32),
        compiler_params=pltpu.CompilerParams(
            dimension_semantics=("arbitrary",),
        ),
    )(x2d, g0, g1, g2, w0, w1, w2)


def kernel(x, E0, W0, E1, W1, E2, W2):
    xf = x.reshape(-1)
    e2p = E2.reshape(-1, 2 * _D2)
    bf = jnp.bfloat16
    g0, g1, g2 = _sc_gather(xf, E0, E1, e2p)
    out = _tc_matmul(xf.reshape(-1, 1), g0, g1, g2,
                     W0.astype(bf), W1.astype(bf), W2.astype(bf))
    return out.reshape(x.shape + (_ED,))
